# Initial kernel scaffold; baseline (speedup 1.0000x reference)
import jax
import jax.numpy as jnp
from jax.experimental import pallas as pl


def kernel(x, edge_index, batch_index, W1, b1, W2, b2):
    raise NotImplementedError("write your pallas kernel here")


# speedup vs baseline: 20.0055x; 20.0055x over previous
import functools

import jax
import jax.numpy as jnp
from jax import lax
from jax.experimental import pallas as pl
from jax.experimental.pallas import tpu as pltpu
from jax.experimental.pallas import tpu_sc as plsc

_N = 10000
_E = 320000
_G = 64
_DIN = 128
_DHID = 64
_DOUT = 8

_NC = 2
_NS = 16
_CHUNK = 128
_NT = 10240
_RPT = _NT // _NS

_TRIPS = -(-_E // (_CHUNK * _NC * _NS))
_EPAD = _TRIPS * _CHUNK * _NC * _NS
_EPT = _TRIPS * _CHUNK


def _sc_mesh():
    return plsc.VectorSubcoreMesh(core_axis_name="c", subcore_axis_name="s")


def _deg_kernel(col_hbm, ones_hbm, zeros_hbm, out_hbm, col_v, ones_v, acc, sem):
    c = lax.axis_index("c")
    s = lax.axis_index("s")
    pltpu.sync_copy(zeros_hbm.at[pl.ds(s * _RPT, _RPT)], acc.at[pl.ds(s * _RPT, _RPT)])
    pltpu.sync_copy(ones_hbm, ones_v)
    plsc.subcore_barrier()
    base0 = (c * _NS + s) * _EPT

    @pl.loop(0, _TRIPS)
    def _(t):
        base = base0 + t * _CHUNK
        pltpu.sync_copy(col_hbm.at[pl.ds(base, _CHUNK)], col_v)
        pltpu.sync_copy(ones_v, acc.at[col_v], add=True)

    plsc.subcore_barrier()
    pltpu.sync_copy(acc.at[pl.ds(s * _RPT, _RPT)], out_hbm.at[c, pl.ds(s * _RPT, _RPT)])


def _make_deg():
    return pl.kernel(
        _deg_kernel,
        out_type=jax.ShapeDtypeStruct((_NC, _NT), jnp.float32),
        mesh=_sc_mesh(),
        scratch_types=[
            pltpu.VMEM((_CHUNK,), jnp.int32),
            pltpu.VMEM((_CHUNK,), jnp.float32),
            pltpu.MemorySpace.VMEM_SHARED((_NT,), jnp.float32),
            pltpu.SemaphoreType.DMA,
        ],
    )


def _make_scatter(d):
    def body(row_hbm, col_hbm, g_hbm, zeros_hbm, out_hbm,
             row_v, col_v, rows_v, acc, sem):
        c = lax.axis_index("c")
        s = lax.axis_index("s")
        pltpu.sync_copy(zeros_hbm.at[pl.ds(s * _RPT, _RPT)],
                        acc.at[pl.ds(s * _RPT, _RPT)])
        plsc.subcore_barrier()
        base0 = (c * _NS + s) * _EPT

        @pl.loop(0, _TRIPS)
        def _(t):
            base = base0 + t * _CHUNK
            pltpu.sync_copy(row_hbm.at[pl.ds(base, _CHUNK)], row_v)
            pltpu.sync_copy(col_hbm.at[pl.ds(base, _CHUNK)], col_v)
            pltpu.async_copy(g_hbm.at[row_v], rows_v, sem).wait()
            pltpu.sync_copy(rows_v, acc.at[col_v], add=True)

        plsc.subcore_barrier()
        pltpu.sync_copy(acc.at[pl.ds(s * _RPT, _RPT)],
                        out_hbm.at[c, pl.ds(s * _RPT, _RPT)])

    return pl.kernel(
        body,
        out_type=jax.ShapeDtypeStruct((_NC, _NT, d), jnp.float32),
        mesh=_sc_mesh(),
        compiler_params=pltpu.CompilerParams(use_tc_tiling_on_sc=False),
        scratch_types=[
            pltpu.VMEM((_CHUNK,), jnp.int32),
            pltpu.VMEM((_CHUNK,), jnp.int32),
            pltpu.VMEM((_CHUNK, d), jnp.float32),
            pltpu.MemorySpace.VMEM_SHARED((_NT, d), jnp.float32),
            pltpu.SemaphoreType.DMA,
        ],
    )


def _mm1_body(x_ref, w1_ref, degt_ref, g1_ref, dis_ref):
    deg = degt_ref[:, 0:1] + degt_ref[:, 1:2] + 1.0
    dis = lax.rsqrt(deg)
    h = jnp.dot(x_ref[...], w1_ref[...], preferred_element_type=jnp.float32)
    g1_ref[...] = h * dis
    dis_ref[...] = dis


def _mm2_body(s1a_ref, s1b_ref, g1_ref, dis_ref, b1_ref, w2_ref, g2_ref):
    dis = dis_ref[...]
    h1 = jnp.tanh(dis * (s1a_ref[...] + s1b_ref[...] + g1_ref[...]) + b1_ref[...])
    z2 = jnp.dot(h1, w2_ref[...], preferred_element_type=jnp.float32)
    g2_ref[...] = z2 * dis


def _tail_body(s2a_ref, s2b_ref, g2_ref, dis_ref, b2_ref, batch_ref, out_ref):
    dis = dis_ref[...]
    h2 = jnp.tanh(dis * (s2a_ref[...] + s2b_ref[...] + g2_ref[...]) + b2_ref[...])
    gid = lax.broadcasted_iota(jnp.int32, (1, _G), 1)
    mask = (batch_ref[...] == gid).astype(jnp.float32)
    dn = (((0,), (0,)), ((), ()))
    sums = lax.dot_general(mask, h2, dn,
                           preferred_element_type=jnp.float32)
    ones = jnp.ones((_N, 1), jnp.float32)
    cnt = lax.dot_general(mask, ones, dn,
                          preferred_element_type=jnp.float32)
    mean = sums / jnp.maximum(cnt, 1.0)
    out_ref[...] = 1.0 / (1.0 + jnp.exp(-mean))


def kernel(x, edge_index, batch_index, W1, b1, W2, b2):
    row = edge_index[0].astype(jnp.int32)
    col = edge_index[1].astype(jnp.int32)
    npad = _EPAD - _E
    pad_r = (jnp.arange(npad, dtype=jnp.int32) * 37) % _N
    pad_c = _N + (jnp.arange(npad, dtype=jnp.int32) % (_NT - _N))
    row_p = jnp.concatenate([row, pad_r])
    col_p = jnp.concatenate([col, pad_c])

    ones128 = jnp.ones((_CHUNK,), jnp.float32)
    zeros1 = jnp.zeros((_NT,), jnp.float32)
    zeros64 = jnp.zeros((_NT, _DHID), jnp.float32)
    zeros8 = jnp.zeros((_NT, _DOUT), jnp.float32)

    degp = _make_deg()(col_p, ones128, zeros1)
    degt = degp.T[:_N]

    g1, dis = pl.pallas_call(
        _mm1_body,
        out_shape=[jax.ShapeDtypeStruct((_N, _DHID), jnp.float32),
                   jax.ShapeDtypeStruct((_N, 1), jnp.float32)],
    )(x, W1, degt)

    s1 = _make_scatter(_DHID)(row_p, col_p, g1, zeros64)

    g2 = pl.pallas_call(
        _mm2_body,
        out_shape=jax.ShapeDtypeStruct((_N, _DOUT), jnp.float32),
    )(s1[0, :_N], s1[1, :_N], g1, dis, b1.reshape(1, _DHID), W2)

    s2 = _make_scatter(_DOUT)(row_p, col_p, g2, zeros8)

    out = pl.pallas_call(
        _tail_body,
        out_shape=jax.ShapeDtypeStruct((_G, _DOUT), jnp.float32),
    )(s2[0, :_N], s2[1, :_N], g2, dis, b2.reshape(1, _DOUT),
      batch_index.astype(jnp.int32).reshape(_N, 1))
    return out


# speedup vs baseline: 50.4917x; 2.5239x over previous
import functools

import jax
import jax.numpy as jnp
from jax import lax
from jax.experimental import pallas as pl
from jax.experimental.pallas import tpu as pltpu
from jax.experimental.pallas import tpu_sc as plsc

_N = 10000
_E = 320000
_G = 64
_DIN = 128
_DHID = 64
_DOUT = 8

_NC = 2
_NS = 16
_CHUNK = 128
_NT = 10240
_RPT = _NT // _NS

_KD = 8
_GROUPS = 10
_TRIPS = _KD * _GROUPS
_EPAD = _TRIPS * _CHUNK * _NC * _NS
_NCHUNKS = _EPAD // _CHUNK


def _sc_mesh():
    return plsc.VectorSubcoreMesh(core_axis_name="c", subcore_axis_name="s")


def _deg_kernel(col_hbm, ones_hbm, zeros_hbm, out_hbm, col2d, ones_v, acc, sem):
    c = lax.axis_index("c")
    s = lax.axis_index("s")
    pltpu.sync_copy(zeros_hbm.at[pl.ds(s * _RPT, _RPT)], acc.at[pl.ds(s * _RPT, _RPT)])
    pltpu.sync_copy(ones_hbm, ones_v)
    trip0 = (c * _NS + s) * _TRIPS
    pltpu.sync_copy(col_hbm.at[pl.ds(trip0, _TRIPS)], col2d)
    plsc.subcore_barrier()

    @pl.loop(0, _GROUPS)
    def _(g):
        descs = []
        for b in range(_KD):
            t = g * _KD + b
            descs.append(pltpu.async_copy(ones_v, acc.at[col2d.at[t]], sem, add=True))
        for d in descs:
            d.wait()

    plsc.subcore_barrier()
    pltpu.sync_copy(acc.at[pl.ds(s * _RPT, _RPT)], out_hbm.at[c, pl.ds(s * _RPT, _RPT)])


def _make_deg():
    return pl.kernel(
        _deg_kernel,
        out_type=jax.ShapeDtypeStruct((_NC, _NT), jnp.float32),
        mesh=_sc_mesh(),
        compiler_params=pltpu.CompilerParams(use_tc_tiling_on_sc=False),
        scratch_types=[
            pltpu.VMEM((_TRIPS, _CHUNK), jnp.int32),
            pltpu.VMEM((_CHUNK,), jnp.float32),
            pltpu.MemorySpace.VMEM_SHARED((_NT,), jnp.float32),
            pltpu.SemaphoreType.DMA,
        ],
    )


def _make_scatter(d):
    def body(row_hbm, col_hbm, g_hbm, zeros_hbm, out_hbm,
             row2d, col2d, rows, acc, gsem, ssem):
        c = lax.axis_index("c")
        s = lax.axis_index("s")
        pltpu.sync_copy(zeros_hbm.at[pl.ds(s * _RPT, _RPT)],
                        acc.at[pl.ds(s * _RPT, _RPT)])
        trip0 = (c * _NS + s) * _TRIPS
        pltpu.sync_copy(row_hbm.at[pl.ds(trip0, _TRIPS)], row2d)
        pltpu.sync_copy(col_hbm.at[pl.ds(trip0, _TRIPS)], col2d)
        plsc.subcore_barrier()

        for b in range(_KD):
            pltpu.async_copy(g_hbm.at[row2d.at[b]], rows.at[b], gsem)

        @pl.loop(0, _GROUPS)
        def _(g):
            t0 = g * _KD
            sdescs = []
            for b in range(_KD):
                t = t0 + b
                pltpu.make_async_copy(g_hbm.at[row2d.at[t]], rows.at[b], gsem).wait()
                sdescs.append(
                    pltpu.async_copy(rows.at[b], acc.at[col2d.at[t]], ssem, add=True))
            for b in range(_KD):
                sdescs[b].wait()
                tn = t0 + _KD + b
                tn = jnp.where(tn >= _TRIPS, tn - _TRIPS, tn)
                pltpu.async_copy(g_hbm.at[row2d.at[tn]], rows.at[b], gsem)

        for b in range(_KD):
            pltpu.make_async_copy(g_hbm.at[row2d.at[b]], rows.at[b], gsem).wait()
        plsc.subcore_barrier()
        pltpu.sync_copy(acc.at[pl.ds(s * _RPT, _RPT)],
                        out_hbm.at[c, pl.ds(s * _RPT, _RPT)])

    return pl.kernel(
        body,
        out_type=jax.ShapeDtypeStruct((_NC, _NT, d), jnp.float32),
        mesh=_sc_mesh(),
        compiler_params=pltpu.CompilerParams(use_tc_tiling_on_sc=False),
        scratch_types=[
            pltpu.VMEM((_TRIPS, _CHUNK), jnp.int32),
            pltpu.VMEM((_TRIPS, _CHUNK), jnp.int32),
            pltpu.VMEM((_KD, _CHUNK, d), jnp.float32),
            pltpu.MemorySpace.VMEM_SHARED((_NT, d), jnp.float32),
            pltpu.SemaphoreType.DMA,
            pltpu.SemaphoreType.DMA,
        ],
    )


def _mm1_body(x_ref, w1_ref, degt_ref, g1_ref, dis_ref):
    deg = degt_ref[:, 0:1] + degt_ref[:, 1:2] + 1.0
    dis = lax.rsqrt(deg)
    h = jnp.dot(x_ref[...], w1_ref[...], preferred_element_type=jnp.float32)
    g1_ref[...] = h * dis
    dis_ref[...] = dis


def _mm2_body(s1_ref, g1_ref, dis_ref, b1_ref, w2_ref, g2_ref):
    dis = dis_ref[...]
    agg = s1_ref[0, :_N, :] + s1_ref[1, :_N, :] + g1_ref[...]
    h1 = jnp.tanh(dis * agg + b1_ref[...])
    z2 = jnp.dot(h1, w2_ref[...], preferred_element_type=jnp.float32)
    g2_ref[...] = z2 * dis


def _tail_body(s2_ref, g2_ref, dis_ref, b2_ref, batch_ref, out_ref):
    dis = dis_ref[...]
    agg = s2_ref[0, :_N, :] + s2_ref[1, :_N, :] + g2_ref[...]
    h2 = jnp.tanh(dis * agg + b2_ref[...])
    gid = lax.broadcasted_iota(jnp.int32, (1, _G), 1)
    mask = (batch_ref[...] == gid).astype(jnp.float32)
    dn = (((0,), (0,)), ((), ()))
    sums = lax.dot_general(mask, h2, dn,
                           preferred_element_type=jnp.float32)
    ones = jnp.ones((_N, 1), jnp.float32)
    cnt = lax.dot_general(mask, ones, dn,
                          preferred_element_type=jnp.float32)
    mean = sums / jnp.maximum(cnt, 1.0)
    out_ref[...] = 1.0 / (1.0 + jnp.exp(-mean))


def kernel(x, edge_index, batch_index, W1, b1, W2, b2):
    row = edge_index[0].astype(jnp.int32)
    col = edge_index[1].astype(jnp.int32)
    npad = _EPAD - _E
    pad_r = (jnp.arange(npad, dtype=jnp.int32) * 37) % _N
    pad_c = _N + (jnp.arange(npad, dtype=jnp.int32) % (_NT - _N))
    row_p = jnp.concatenate([row, pad_r]).reshape(_NCHUNKS, _CHUNK)
    col_p = jnp.concatenate([col, pad_c]).reshape(_NCHUNKS, _CHUNK)

    ones128 = jnp.ones((_CHUNK,), jnp.float32)
    zeros1 = jnp.zeros((_NT,), jnp.float32)
    zeros64 = jnp.zeros((_NT, _DHID), jnp.float32)
    zeros8 = jnp.zeros((_NT, _DOUT), jnp.float32)

    degp = _make_deg()(col_p, ones128, zeros1)
    degt = degp.T[:_N]

    g1, dis = pl.pallas_call(
        _mm1_body,
        out_shape=[jax.ShapeDtypeStruct((_N, _DHID), jnp.float32),
                   jax.ShapeDtypeStruct((_N, 1), jnp.float32)],
    )(x, W1, degt)

    s1 = _make_scatter(_DHID)(row_p, col_p, g1, zeros64)

    g2 = pl.pallas_call(
        _mm2_body,
        out_shape=jax.ShapeDtypeStruct((_N, _DOUT), jnp.float32),
    )(s1, g1, dis, b1.reshape(1, _DHID), W2)

    s2 = _make_scatter(_DOUT)(row_p, col_p, g2, zeros8)

    out = pl.pallas_call(
        _tail_body,
        out_shape=jax.ShapeDtypeStruct((_G, _DOUT), jnp.float32),
    )(s2, g2, dis, b2.reshape(1, _DOUT),
      batch_index.astype(jnp.int32).reshape(_N, 1))
    return out


# speedup vs baseline: 51.0680x; 1.0114x over previous
import functools

import jax
import jax.numpy as jnp
from jax import lax
from jax.experimental import pallas as pl
from jax.experimental.pallas import tpu as pltpu
from jax.experimental.pallas import tpu_sc as plsc

_N = 10000
_E = 320000
_G = 64
_DIN = 128
_DHID = 64
_DOUT = 8

_NC = 2
_NS = 16
_CHUNK = 128
_NT = 10240
_RPT = _NT // _NS

_TRIPS = 80
_EPAD = _TRIPS * _CHUNK * _NC * _NS
_NCHUNKS = _EPAD // _CHUNK


def _sc_mesh():
    return plsc.VectorSubcoreMesh(core_axis_name="c", subcore_axis_name="s")


def _deg_kernel(col_hbm, ones_hbm, zeros_hbm, out_hbm, col2d, ones_v, acc, sem):
    c = lax.axis_index("c")
    s = lax.axis_index("s")
    pltpu.sync_copy(zeros_hbm.at[pl.ds(s * _RPT, _RPT)], acc.at[pl.ds(s * _RPT, _RPT)])
    pltpu.sync_copy(ones_hbm, ones_v)
    trip0 = (c * _NS + s) * _TRIPS
    pltpu.sync_copy(col_hbm.at[pl.ds(trip0, _TRIPS)], col2d)
    plsc.subcore_barrier()

    @pl.loop(0, _TRIPS // 16)
    def _(g):
        descs = []
        for b in range(16):
            t = g * 16 + b
            descs.append(pltpu.async_copy(ones_v, acc.at[col2d.at[t]], sem, add=True))
        for d in descs:
            d.wait()

    plsc.subcore_barrier()
    pltpu.sync_copy(acc.at[pl.ds(s * _RPT, _RPT)], out_hbm.at[c, pl.ds(s * _RPT, _RPT)])


def _make_deg():
    return pl.kernel(
        _deg_kernel,
        out_type=jax.ShapeDtypeStruct((_NC, _NT), jnp.float32),
        mesh=_sc_mesh(),
        compiler_params=pltpu.CompilerParams(use_tc_tiling_on_sc=False),
        scratch_types=[
            pltpu.VMEM((_TRIPS, _CHUNK), jnp.int32),
            pltpu.VMEM((_CHUNK,), jnp.float32),
            pltpu.MemorySpace.VMEM_SHARED((_NT,), jnp.float32),
            pltpu.SemaphoreType.DMA,
        ],
    )


def _make_scatter(d, kd):
    groups = _TRIPS // kd

    def body(row_hbm, col_hbm, g_hbm, zeros_hbm, out_hbm,
             row2d, col2d, rows, acc, gsem, ssem):
        c = lax.axis_index("c")
        s = lax.axis_index("s")
        pltpu.sync_copy(zeros_hbm.at[pl.ds(s * _RPT, _RPT)],
                        acc.at[pl.ds(s * _RPT, _RPT)])
        trip0 = (c * _NS + s) * _TRIPS
        pltpu.sync_copy(row_hbm.at[pl.ds(trip0, _TRIPS)], row2d)
        pltpu.sync_copy(col_hbm.at[pl.ds(trip0, _TRIPS)], col2d)
        plsc.subcore_barrier()

        for b in range(kd):
            pltpu.async_copy(g_hbm.at[row2d.at[b]], rows.at[b], gsem)

        @pl.loop(0, groups)
        def _(g):
            t0 = g * kd
            sdescs = []
            for b in range(kd):
                t = t0 + b
                pltpu.make_async_copy(g_hbm.at[row2d.at[t]], rows.at[b], gsem).wait()
                sdescs.append(
                    pltpu.async_copy(rows.at[b], acc.at[col2d.at[t]], ssem, add=True))
            for b in range(kd):
                sdescs[b].wait()
                tn = t0 + kd + b
                tn = jnp.where(tn >= _TRIPS, tn - _TRIPS, tn)
                pltpu.async_copy(g_hbm.at[row2d.at[tn]], rows.at[b], gsem)

        for b in range(kd):
            pltpu.make_async_copy(g_hbm.at[row2d.at[b]], rows.at[b], gsem).wait()
        plsc.subcore_barrier()
        pltpu.sync_copy(acc.at[pl.ds(s * _RPT, _RPT)],
                        out_hbm.at[c, pl.ds(s * _RPT, _RPT)])

    return pl.kernel(
        body,
        out_type=jax.ShapeDtypeStruct((_NC, _NT, d), jnp.float32),
        mesh=_sc_mesh(),
        compiler_params=pltpu.CompilerParams(use_tc_tiling_on_sc=False),
        scratch_types=[
            pltpu.VMEM((_TRIPS, _CHUNK), jnp.int32),
            pltpu.VMEM((_TRIPS, _CHUNK), jnp.int32),
            pltpu.VMEM((kd, _CHUNK, d), jnp.float32),
            pltpu.MemorySpace.VMEM_SHARED((_NT, d), jnp.float32),
            pltpu.SemaphoreType.DMA,
            pltpu.SemaphoreType.DMA,
        ],
    )


def _mm1_body(x_ref, w1_ref, degt_ref, g1_ref, dis_ref):
    deg = degt_ref[:, 0:1] + degt_ref[:, 1:2] + 1.0
    dis = lax.rsqrt(deg)
    h = jnp.dot(x_ref[...], w1_ref[...], preferred_element_type=jnp.float32)
    g1_ref[...] = h * dis
    dis_ref[...] = dis


def _mm2_body(s1_ref, g1_ref, dis_ref, b1_ref, w2_ref, g2_ref):
    dis = dis_ref[...]
    agg = s1_ref[0, :_N, :] + s1_ref[1, :_N, :] + g1_ref[...]
    h1 = jnp.tanh(dis * agg + b1_ref[...])
    z2 = jnp.dot(h1, w2_ref[...], preferred_element_type=jnp.float32)
    g2_ref[...] = z2 * dis


def _tail_body(s2_ref, g2_ref, dis_ref, b2_ref, batch_ref, out_ref):
    dis = dis_ref[...]
    agg = s2_ref[0, :_N, :] + s2_ref[1, :_N, :] + g2_ref[...]
    h2 = jnp.tanh(dis * agg + b2_ref[...])
    gid = lax.broadcasted_iota(jnp.int32, (1, _G), 1)
    mask = (batch_ref[...] == gid).astype(jnp.float32)
    dn = (((0,), (0,)), ((), ()))
    sums = lax.dot_general(mask, h2, dn,
                           preferred_element_type=jnp.float32)
    ones = jnp.ones((_N, 1), jnp.float32)
    cnt = lax.dot_general(mask, ones, dn,
                          preferred_element_type=jnp.float32)
    mean = sums / jnp.maximum(cnt, 1.0)
    out_ref[...] = 1.0 / (1.0 + jnp.exp(-mean))


def kernel(x, edge_index, batch_index, W1, b1, W2, b2):
    row = edge_index[0].astype(jnp.int32)
    col = edge_index[1].astype(jnp.int32)
    npad = _EPAD - _E
    pad_r = (jnp.arange(npad, dtype=jnp.int32) * 37) % _N
    pad_c = _N + (jnp.arange(npad, dtype=jnp.int32) % (_NT - _N))
    row_p = jnp.concatenate([row, pad_r]).reshape(_NCHUNKS, _CHUNK)
    col_p = jnp.concatenate([col, pad_c]).reshape(_NCHUNKS, _CHUNK)

    ones128 = jnp.ones((_CHUNK,), jnp.float32)
    zeros1 = jnp.zeros((_NT,), jnp.float32)
    zeros64 = jnp.zeros((_NT, _DHID), jnp.float32)
    zeros8 = jnp.zeros((_NT, _DOUT), jnp.float32)

    degp = _make_deg()(col_p, ones128, zeros1)
    degt = degp.T[:_N]

    g1, dis = pl.pallas_call(
        _mm1_body,
        out_shape=[jax.ShapeDtypeStruct((_N, _DHID), jnp.float32),
                   jax.ShapeDtypeStruct((_N, 1), jnp.float32)],
    )(x, W1, degt)

    s1 = _make_scatter(_DHID, 8)(row_p, col_p, g1, zeros64)

    g2 = pl.pallas_call(
        _mm2_body,
        out_shape=jax.ShapeDtypeStruct((_N, _DOUT), jnp.float32),
    )(s1, g1, dis, b1.reshape(1, _DHID), W2)

    s2 = _make_scatter(_DOUT, 16)(row_p, col_p, g2, zeros8)

    out = pl.pallas_call(
        _tail_body,
        out_shape=jax.ShapeDtypeStruct((_G, _DOUT), jnp.float32),
    )(s2, g2, dis, b2.reshape(1, _DOUT),
      batch_index.astype(jnp.int32).reshape(_N, 1))
    return out


# speedup vs baseline: 52.3724x; 1.0255x over previous
import functools

import jax
import jax.numpy as jnp
from jax import lax
from jax.experimental import pallas as pl
from jax.experimental.pallas import tpu as pltpu
from jax.experimental.pallas import tpu_sc as plsc

_N = 10000
_E = 320000
_G = 64
_DIN = 128
_DHID = 64
_DOUT = 8

_NC = 2
_NS = 16
_CHUNK = 128
_NT = 10240
_RPT = _NT // _NS

_TRIPS = 80
_EPAD = _TRIPS * _CHUNK * _NC * _NS
_NCHUNKS = _EPAD // _CHUNK


def _sc_mesh():
    return plsc.VectorSubcoreMesh(core_axis_name="c", subcore_axis_name="s")


def _deg_kernel(col_hbm, ones_hbm, zeros_hbm, out_hbm, col2d, ones_v, acc, sem):
    c = lax.axis_index("c")
    s = lax.axis_index("s")
    pltpu.sync_copy(zeros_hbm.at[pl.ds(s * _RPT, _RPT)], acc.at[pl.ds(s * _RPT, _RPT)])
    pltpu.sync_copy(ones_hbm, ones_v)
    trip0 = (c * _NS + s) * _TRIPS
    pltpu.sync_copy(col_hbm.at[pl.ds(trip0, _TRIPS)], col2d)
    plsc.subcore_barrier()

    @pl.loop(0, _TRIPS // 16)
    def _(g):
        descs = []
        for b in range(16):
            t = g * 16 + b
            descs.append(pltpu.async_copy(ones_v, acc.at[col2d.at[t]], sem, add=True))
        for d in descs:
            d.wait()

    plsc.subcore_barrier()
    pltpu.sync_copy(acc.at[pl.ds(s * _RPT, _RPT)], out_hbm.at[c, pl.ds(s * _RPT, _RPT)])


def _make_deg():
    return pl.kernel(
        _deg_kernel,
        out_type=jax.ShapeDtypeStruct((_NC, _NT), jnp.float32),
        mesh=_sc_mesh(),
        compiler_params=pltpu.CompilerParams(use_tc_tiling_on_sc=False),
        scratch_types=[
            pltpu.VMEM((_TRIPS, _CHUNK), jnp.int32),
            pltpu.VMEM((_CHUNK,), jnp.float32),
            pltpu.MemorySpace.VMEM_SHARED((_NT,), jnp.float32),
            pltpu.SemaphoreType.DMA,
        ],
    )


def _make_scatter(d, kd):
    groups = _TRIPS // kd

    def body(row_hbm, col_hbm, g_hbm, zeros_hbm, out_hbm,
             row2d, col2d, rows, acc, gsem, ssem):
        c = lax.axis_index("c")
        s = lax.axis_index("s")
        pltpu.sync_copy(zeros_hbm.at[pl.ds(s * _RPT, _RPT)],
                        acc.at[pl.ds(s * _RPT, _RPT)])
        trip0 = (c * _NS + s) * _TRIPS
        pltpu.sync_copy(row_hbm.at[pl.ds(trip0, _TRIPS)], row2d)
        pltpu.sync_copy(col_hbm.at[pl.ds(trip0, _TRIPS)], col2d)
        plsc.subcore_barrier()

        for b in range(kd):
            pltpu.async_copy(g_hbm.at[row2d.at[b]], rows.at[b], gsem)

        @pl.loop(0, groups)
        def _(g):
            t0 = g * kd
            sdescs = []
            for b in range(kd):
                t = t0 + b
                pltpu.make_async_copy(g_hbm.at[row2d.at[t]], rows.at[b], gsem).wait()
                sdescs.append(
                    pltpu.async_copy(rows.at[b], acc.at[col2d.at[t]], ssem, add=True))
            for b in range(kd):
                sdescs[b].wait()
                tn = t0 + kd + b
                tn = jnp.where(tn >= _TRIPS, tn - _TRIPS, tn)
                pltpu.async_copy(g_hbm.at[row2d.at[tn]], rows.at[b], gsem)

        for b in range(kd):
            pltpu.make_async_copy(g_hbm.at[row2d.at[b]], rows.at[b], gsem).wait()
        plsc.subcore_barrier()
        pltpu.sync_copy(acc.at[pl.ds(s * _RPT, _RPT)],
                        out_hbm.at[c, pl.ds(s * _RPT, _RPT)])

    return pl.kernel(
        body,
        out_type=jax.ShapeDtypeStruct((_NC, _NT, d), jnp.float32),
        mesh=_sc_mesh(),
        compiler_params=pltpu.CompilerParams(use_tc_tiling_on_sc=False),
        scratch_types=[
            pltpu.VMEM((_TRIPS, _CHUNK), jnp.int32),
            pltpu.VMEM((_TRIPS, _CHUNK), jnp.int32),
            pltpu.VMEM((kd, _CHUNK, d), jnp.float32),
            pltpu.MemorySpace.VMEM_SHARED((_NT, d), jnp.float32),
            pltpu.SemaphoreType.DMA,
            pltpu.SemaphoreType.DMA,
        ],
    )


def _mm1_body(x_ref, w1_ref, degp_ref, g1_ref, dis_ref):
    dn = (((0,), (0,)), ((), ()))
    degc = lax.dot_general(degp_ref[...], jnp.ones((_NC, 1), jnp.float32), dn,
                           preferred_element_type=jnp.float32)
    deg = degc[:_N] + 1.0
    dis = lax.rsqrt(deg)
    h = jnp.dot(x_ref[...], w1_ref[...], preferred_element_type=jnp.float32)
    g1_ref[...] = h * dis
    dis_ref[...] = dis


def _mm2_body(s1_ref, g1_ref, dis_ref, b1_ref, w2_ref, g2_ref):
    dis = dis_ref[...]
    agg = s1_ref[0, :_N, :] + s1_ref[1, :_N, :] + g1_ref[...]
    h1 = jnp.tanh(dis * agg + b1_ref[...])
    z2 = jnp.dot(h1, w2_ref[...], preferred_element_type=jnp.float32)
    g2_ref[...] = z2 * dis


def _tail_body(s2_ref, g2_ref, dis_ref, b2_ref, batch_ref, out_ref):
    dis = dis_ref[...]
    agg = s2_ref[0, :_N, :] + s2_ref[1, :_N, :] + g2_ref[...]
    h2 = jnp.tanh(dis * agg + b2_ref[...])
    gid = lax.broadcasted_iota(jnp.int32, (1, _G), 1)
    mask = (batch_ref[...] == gid).astype(jnp.float32)
    dn = (((0,), (0,)), ((), ()))
    sums = lax.dot_general(mask, h2, dn,
                           preferred_element_type=jnp.float32)
    ones = jnp.ones((_N, 1), jnp.float32)
    cnt = lax.dot_general(mask, ones, dn,
                          preferred_element_type=jnp.float32)
    mean = sums / jnp.maximum(cnt, 1.0)
    out_ref[...] = 1.0 / (1.0 + jnp.exp(-mean))


def kernel(x, edge_index, batch_index, W1, b1, W2, b2):
    row = edge_index[0].astype(jnp.int32)
    col = edge_index[1].astype(jnp.int32)
    npad = _EPAD - _E
    pad_r = (jnp.arange(npad, dtype=jnp.int32) * 37) % _N
    pad_c = _N + (jnp.arange(npad, dtype=jnp.int32) % (_NT - _N))
    row_p = jnp.concatenate([row, pad_r]).reshape(_NCHUNKS, _CHUNK)
    col_p = jnp.concatenate([col, pad_c]).reshape(_NCHUNKS, _CHUNK)

    ones128 = jnp.ones((_CHUNK,), jnp.float32)
    zeros1 = jnp.zeros((_NT,), jnp.float32)
    zeros64 = jnp.zeros((_NT, _DHID), jnp.float32)
    zeros8 = jnp.zeros((_NT, _DOUT), jnp.float32)

    degp = _make_deg()(col_p, ones128, zeros1)

    g1, dis = pl.pallas_call(
        _mm1_body,
        out_shape=[jax.ShapeDtypeStruct((_N, _DHID), jnp.float32),
                   jax.ShapeDtypeStruct((_N, 1), jnp.float32)],
    )(x, W1, degp)

    s1 = _make_scatter(_DHID, 8)(row_p, col_p, g1, zeros64)

    g2 = pl.pallas_call(
        _mm2_body,
        out_shape=jax.ShapeDtypeStruct((_N, _DOUT), jnp.float32),
    )(s1, g1, dis, b1.reshape(1, _DHID), W2)

    s2 = _make_scatter(_DOUT, 16)(row_p, col_p, g2, zeros8)

    out = pl.pallas_call(
        _tail_body,
        out_shape=jax.ShapeDtypeStruct((_G, _DOUT), jnp.float32),
    )(s2, g2, dis, b2.reshape(1, _DOUT),
      batch_index.astype(jnp.int32).reshape(_N, 1))
    return out
